# Initial kernel scaffold; baseline (speedup 1.0000x reference)
import jax
import jax.numpy as jnp
from jax.experimental import pallas as pl


def kernel(graph_nodes, graph_edge_links, graph_edges, W1, b1, W2, b2, W3, b3, W4, b4, Wl1, bl1, Wl2, bl2, Wls, bls):
    raise NotImplementedError("write your pallas kernel here")


# speedup vs baseline: 1.5391x; 1.5391x over previous
import functools

import jax
import jax.numpy as jnp
from jax.experimental import pallas as pl
from jax.experimental.pallas import tpu as pltpu


def _leaky(v):
    return jnp.where(v >= 0, v, 0.01 * v)


def _head_body(x_ref, wl1_ref, bl1_ref, wl2_ref, bl2_ref, wls_ref, bls_ref,
               out_ref, acc_ref, *, blocks_per_graph, num_blocks):
    i = pl.program_id(0)

    @pl.when(i == 0)
    def _init():
        acc_ref[...] = jnp.zeros_like(acc_ref)

    x = x_ref[...]
    t = _leaky(jnp.dot(x, wl1_ref[...], preferred_element_type=jnp.float32)
               + bl1_ref[...][None, :])
    u = jnp.dot(t, wl2_ref[...], preferred_element_type=jnp.float32) \
        + bl2_ref[...][None, :]
    usum = jnp.sum(u, axis=0, keepdims=True)
    g = i // blocks_per_graph
    acc_ref[pl.ds(g, 1), :64] += usum

    @pl.when(i == num_blocks - 1)
    def _fin():
        pooled = acc_ref[:, :64]
        res = jnp.dot(pooled, wls_ref[...], preferred_element_type=jnp.float32)
        out_ref[...] = res[:4, :] + bls_ref[...][None, :]


def _head(x, Wl1, bl1, Wl2, bl2, Wls, bls, batch_size, n_per_graph):
    num_nodes = x.shape[0]
    br = 2000
    num_blocks = num_nodes // br
    blocks_per_graph = n_per_graph // br
    body = functools.partial(_head_body, blocks_per_graph=blocks_per_graph,
                             num_blocks=num_blocks)
    return pl.pallas_call(
        body,
        grid=(num_blocks,),
        in_specs=[
            pl.BlockSpec((br, 128), lambda i: (i, 0)),
            pl.BlockSpec((128, 128), lambda i: (0, 0)),
            pl.BlockSpec((128,), lambda i: (0,)),
            pl.BlockSpec((128, 64), lambda i: (0, 0)),
            pl.BlockSpec((64,), lambda i: (0,)),
            pl.BlockSpec((64, 1), lambda i: (0, 0)),
            pl.BlockSpec((1,), lambda i: (0,)),
        ],
        out_specs=pl.BlockSpec((batch_size, 1), lambda i: (0, 0)),
        out_shape=jax.ShapeDtypeStruct((batch_size, 1), jnp.float32),
        scratch_shapes=[pltpu.VMEM((8, 64), jnp.float32)],
    )(x, Wl1, bl1, Wl2, bl2, Wls, bls)


def kernel(graph_nodes, graph_edge_links, graph_edges, W1, b1, W2, b2, W3, b3,
           W4, b4, Wl1, bl1, Wl2, bl2, Wls, bls):
    B, N, D = graph_nodes.shape
    num_nodes = B * N
    offsets = (jnp.arange(B, dtype=graph_edge_links.dtype) * N)[:, None, None]
    ei = (graph_edge_links + offsets).transpose(1, 0, 2).reshape(2, -1)
    src, dst = ei[0], ei[1]
    x = graph_nodes.reshape(num_nodes, D)

    deg = jnp.zeros((num_nodes,), jnp.float32).at[dst].add(1.0) + 1.0
    dis = jax.lax.rsqrt(deg)

    for (W, b) in ((W1, b1), (W2, b2), (W3, b3), (W4, b4)):
        g = (x @ W) * dis[:, None]
        msg = jnp.zeros_like(g).at[dst].add(g[src])
        x = _leaky(dis[:, None] * (msg + g) + b)

    return _head(x, Wl1, bl1, Wl2, bl2, Wls, bls, B, N)


# speedup vs baseline: 6.6939x; 4.3492x over previous
import jax
import jax.numpy as jnp
from jax import lax
from jax.experimental import pallas as pl
from jax.experimental.pallas import tpu as pltpu
from jax.experimental.pallas import tpu_sc as plsc

NC = 2
NS = 16
L = 16
K = 128
CF = 16
NCH = 128 // CF
CPS = NCH // NC
NBUF = 4


def _leaky(v):
    return jnp.where(v >= 0, v, 0.01 * v)


def _ceil_to(x, m):
    return (x + m - 1) // m * m


def _fill_rows(ref, nrows, ncols, val, dtype):
    vec = jnp.full((L,), val, dtype)

    def body(i, _):
        for j in range(ncols // L):
            ref[i, pl.ds(j * L, L)] = vec
        return 0

    lax.fori_loop(0, nrows, body, 0)


def _acc_rows(num_nodes):
    return _ceil_to(num_nodes + K, NS * 8)


def _sc_deg(dst32, num_nodes):
    nb = dst32.shape[1]
    acc_rows = _acc_rows(num_nodes)
    zpt = acc_rows // NS
    mesh = plsc.VectorSubcoreMesh(core_axis_name="c", subcore_axis_name="s",
                                  num_cores=NC, num_subcores=NS)

    def body(dst_hbm, out_hbm, dst_v, ones_v, zbuf_v, acc_sh, *sems):
        cid = lax.axis_index("c")
        sid = lax.axis_index("s")
        wid = cid * NS + sid
        _fill_rows(ones_v, K, L, 1.0, jnp.float32)
        _fill_rows(zbuf_v, K, L, 0.0, jnp.float32)
        off = 0
        while off < zpt:
            step = min(K, zpt - off)
            pltpu.sync_copy(zbuf_v.at[pl.ds(0, step)],
                            acc_sh.at[pl.ds(sid * zpt + off, step)])
            off += step
        pltpu.sync_copy(dst_hbm.at[wid], dst_v)
        plsc.subcore_barrier()

        nq = nb // NBUF

        def quad(m, _):
            j0 = m * NBUF
            for b in range(NBUF):
                pltpu.async_copy(ones_v, acc_sh.at[dst_v.at[j0 + b]], sems[b],
                                 add=True)
            for b in range(NBUF):
                pltpu.make_async_copy(ones_v, acc_sh.at[dst_v.at[j0 + b]],
                                      sems[b]).wait()
            return 0

        lax.fori_loop(0, nq, quad, 0)
        for j in range(nq * NBUF, nb):
            pltpu.sync_copy(ones_v, acc_sh.at[dst_v.at[j]], add=True)
        plsc.subcore_barrier()
        pltpu.sync_copy(acc_sh.at[pl.ds(sid * zpt, zpt)],
                        out_hbm.at[cid, pl.ds(sid * zpt, zpt)])

    return pl.kernel(
        body,
        out_type=jax.ShapeDtypeStruct((NC, acc_rows, L), jnp.float32),
        mesh=mesh,
        scratch_types=[
            pltpu.VMEM((nb, K), jnp.int32),
            pltpu.VMEM((K, L), jnp.float32),
            pltpu.VMEM((K, L), jnp.float32),
            pltpu.VMEM_SHARED((acc_rows, L), jnp.float32),
        ] + [pltpu.SemaphoreType.DMA] * NBUF,
        compiler_params=pltpu.CompilerParams(use_tc_tiling_on_sc=False),
    )(dst32)


def _sc_msg(g_flat, src16, dst16, num_nodes):
    nb = src16.shape[1]
    nbp = nb + NBUF
    acc_rows = _acc_rows(num_nodes)
    zpt = acc_rows // NS
    mesh = plsc.VectorSubcoreMesh(core_axis_name="c", subcore_axis_name="s",
                                  num_cores=NC, num_subcores=NS)

    def body(g_hbm, src_hbm, dst_hbm, out_hbm, idx_v, dst_v, r0, r1, r2, r3,
             zbuf_v, acc_sh, sg0, sg1, sg2, sg3, ss0, ss1, ss2, ss3):
        rows = (r0, r1, r2, r3)
        sg = (sg0, sg1, sg2, sg3)
        ss = (ss0, ss1, ss2, ss3)
        cid = lax.axis_index("c")
        sid = lax.axis_index("s")
        _fill_rows(zbuf_v, K, CF, 0.0, jnp.float32)
        pltpu.sync_copy(src_hbm.at[sid], idx_v.at[pl.ds(0, nb)])
        _fill_rows(idx_v.at[pl.ds(nb, NBUF)], NBUF, K, 0, jnp.int32)
        pltpu.sync_copy(dst_hbm.at[sid], dst_v)

        def shift_idx(delta):
            def sbody(i, _):
                for j in range(K // L):
                    sl = pl.ds(j * L, L)
                    idx_v[i, sl] = idx_v[i, sl] + delta
                return 0
            lax.fori_loop(0, nbp, sbody, 0)

        def gather(j, b):
            return pltpu.async_copy(g_hbm.at[idx_v.at[j]], rows[b], sg[b])

        def gather_wait(j, b):
            pltpu.make_async_copy(g_hbm.at[idx_v.at[j]], rows[b], sg[b]).wait()

        def scatter(j, b):
            return pltpu.async_copy(rows[b], acc_sh.at[dst_v.at[j]], ss[b],
                                    add=True)

        def scatter_wait(j, b):
            pltpu.make_async_copy(rows[b], acc_sh.at[dst_v.at[j]],
                                  ss[b]).wait()

        for cc in range(CPS):
            if cc == 0:
                shift_idx(cid * jnp.int32(CPS * num_nodes))
            else:
                shift_idx(jnp.int32(num_nodes))
            off = 0
            while off < zpt:
                step = min(K, zpt - off)
                pltpu.sync_copy(zbuf_v.at[pl.ds(0, step)],
                                acc_sh.at[pl.ds(sid * zpt + off, step)])
                off += step
            plsc.subcore_barrier()

            for b in range(NBUF):
                gather(b, b)
            nq = nb // NBUF

            def quad(m, _):
                j0 = m * NBUF
                for b in range(NBUF):
                    gather_wait(j0 + b, b)
                    scatter(j0 + b, b)
                for b in range(NBUF):
                    scatter_wait(j0 + b, b)
                    gather(j0 + NBUF + b, b)
                return 0

            lax.fori_loop(0, nq, quad, 0)
            for j in range(nq * NBUF, nb):
                b = j % NBUF
                gather_wait(j, b)
                scatter(j, b)
                scatter_wait(j, b)
            for b in range(NBUF):
                j = nq * NBUF + b
                if j >= nb:
                    gather_wait(j, b)

            plsc.subcore_barrier()
            out_base = (cid * CPS + cc) * acc_rows + sid * zpt
            pltpu.sync_copy(acc_sh.at[pl.ds(sid * zpt, zpt)],
                            out_hbm.at[pl.ds(out_base, zpt)])
            plsc.subcore_barrier()

    return pl.kernel(
        body,
        out_type=jax.ShapeDtypeStruct((NCH * acc_rows, CF), jnp.float32),
        mesh=mesh,
        scratch_types=[
            pltpu.VMEM((nbp, K), jnp.int32),
            pltpu.VMEM((nb, K), jnp.int32),
            pltpu.VMEM((K, CF), jnp.float32),
            pltpu.VMEM((K, CF), jnp.float32),
            pltpu.VMEM((K, CF), jnp.float32),
            pltpu.VMEM((K, CF), jnp.float32),
            pltpu.VMEM((K, CF), jnp.float32),
            pltpu.VMEM_SHARED((acc_rows, CF), jnp.float32),
        ] + [pltpu.SemaphoreType.DMA] * (2 * NBUF),
        compiler_params=pltpu.CompilerParams(use_tc_tiling_on_sc=False),
    )(g_flat, src16, dst16)


_BR = 2000


def _tc_layer1(x, W1, deg, num_nodes):
    nblk = num_nodes // _BR

    def body(x_ref, w_ref, deg_ref, g_ref, dis_ref):
        d = deg_ref[...]
        deg_tot = d[0, :, 0:1] + d[1, :, 0:1] + 1.0
        dis = lax.rsqrt(deg_tot)
        dis_ref[...] = dis
        h = jnp.dot(x_ref[...], w_ref[...], preferred_element_type=jnp.float32)
        g = h * dis
        for c in range(NCH):
            g_ref[c, :, :] = g[:, c * CF:(c + 1) * CF]

    return pl.pallas_call(
        body,
        grid=(nblk,),
        in_specs=[
            pl.BlockSpec((_BR, 128), lambda i: (i, 0)),
            pl.BlockSpec((128, 128), lambda i: (0, 0)),
            pl.BlockSpec((NC, _BR, L), lambda i: (0, i, 0)),
        ],
        out_specs=[
            pl.BlockSpec((NCH, _BR, CF), lambda i: (0, i, 0)),
            pl.BlockSpec((_BR, 1), lambda i: (i, 0)),
        ],
        out_shape=[
            jax.ShapeDtypeStruct((NCH, num_nodes, CF), jnp.float32),
            jax.ShapeDtypeStruct((num_nodes, 1), jnp.float32),
        ],
    )(x, W1, deg)


def _tc_layer(msg, g, dis, b, W, num_nodes):
    nblk = num_nodes // _BR

    def body(msg_ref, g_ref, dis_ref, b_ref, w_ref, gn_ref):
        dis = dis_ref[...]
        acc = jnp.zeros((_BR, 128), jnp.float32)
        for c in range(NCH):
            y = _leaky(dis * (msg_ref[c, :, :] + g_ref[c, :, :])
                       + b_ref[c * CF:(c + 1) * CF][None, :])
            acc += jnp.dot(y, w_ref[c * CF:(c + 1) * CF, :],
                           preferred_element_type=jnp.float32)
        gn = acc * dis
        for c in range(NCH):
            gn_ref[c, :, :] = gn[:, c * CF:(c + 1) * CF]

    return pl.pallas_call(
        body,
        grid=(nblk,),
        in_specs=[
            pl.BlockSpec((NCH, _BR, CF), lambda i: (0, i, 0)),
            pl.BlockSpec((NCH, _BR, CF), lambda i: (0, i, 0)),
            pl.BlockSpec((_BR, 1), lambda i: (i, 0)),
            pl.BlockSpec((128,), lambda i: (0,)),
            pl.BlockSpec((128, 128), lambda i: (0, 0)),
        ],
        out_specs=pl.BlockSpec((NCH, _BR, CF), lambda i: (0, i, 0)),
        out_shape=jax.ShapeDtypeStruct((NCH, num_nodes, CF), jnp.float32),
    )(msg, g, dis, b, W)


def _tc_head(msg, g, dis, b4, Wl1, bl1, Wl2, bl2, Wls, bls, batch, n):
    num_nodes = batch * n
    nblk = num_nodes // _BR
    bpg = n // _BR

    def body(msg_ref, g_ref, dis_ref, b_ref, wl1_ref, bl1_ref, wl2_ref,
             bl2_ref, wls_ref, bls_ref, out_ref, acc_ref):
        i = pl.program_id(0)

        @pl.when(i == 0)
        def _init():
            acc_ref[...] = jnp.zeros_like(acc_ref)

        dis = dis_ref[...]
        xs = [
            _leaky(dis * (msg_ref[c, :, :] + g_ref[c, :, :])
                   + b_ref[c * CF:(c + 1) * CF][None, :])
            for c in range(NCH)
        ]
        x = jnp.concatenate(xs, axis=1)
        t = _leaky(jnp.dot(x, wl1_ref[...], preferred_element_type=jnp.float32)
                   + bl1_ref[...][None, :])
        u = jnp.dot(t, wl2_ref[...], preferred_element_type=jnp.float32) \
            + bl2_ref[...][None, :]
        usum = jnp.sum(u, axis=0, keepdims=True)
        gidx = i // bpg
        acc_ref[pl.ds(gidx, 1), :] += usum

        @pl.when(i == nblk - 1)
        def _fin():
            pooled = acc_ref[...]
            res = jnp.dot(pooled, wls_ref[...],
                          preferred_element_type=jnp.float32)
            out_ref[...] = res[:batch, :] + bls_ref[...][None, :]

    return pl.pallas_call(
        body,
        grid=(nblk,),
        in_specs=[
            pl.BlockSpec((NCH, _BR, CF), lambda i: (0, i, 0)),
            pl.BlockSpec((NCH, _BR, CF), lambda i: (0, i, 0)),
            pl.BlockSpec((_BR, 1), lambda i: (i, 0)),
            pl.BlockSpec((128,), lambda i: (0,)),
            pl.BlockSpec((128, 128), lambda i: (0, 0)),
            pl.BlockSpec((128,), lambda i: (0,)),
            pl.BlockSpec((128, 64), lambda i: (0, 0)),
            pl.BlockSpec((64,), lambda i: (0,)),
            pl.BlockSpec((64, 1), lambda i: (0, 0)),
            pl.BlockSpec((1,), lambda i: (0,)),
        ],
        out_specs=pl.BlockSpec((batch, 1), lambda i: (0, 0)),
        out_shape=jax.ShapeDtypeStruct((batch, 1), jnp.float32),
        scratch_shapes=[pltpu.VMEM((8, 64), jnp.float32)],
    )(msg, g, dis, b4, Wl1, bl1, Wl2, bl2, Wls, bls)


def kernel(graph_nodes, graph_edge_links, graph_edges, W1, b1, W2, b2, W3, b3,
           W4, b4, Wl1, bl1, Wl2, bl2, Wls, bls):
    B, N, D = graph_nodes.shape
    E = graph_edge_links.shape[2]
    num_nodes = B * N
    ET = B * E

    offsets = (jnp.arange(B, dtype=graph_edge_links.dtype) * N)[:, None, None]
    ei = (graph_edge_links + offsets).transpose(1, 0, 2).reshape(2, -1)
    src, dst = ei[0].astype(jnp.int32), ei[1].astype(jnp.int32)
    x = graph_nodes.reshape(num_nodes, D)

    ep32 = _ceil_to(ET, NC * NS * K)
    dst32 = jnp.concatenate(
        [dst, jnp.full((ep32 - ET,), num_nodes, jnp.int32)]
    ).reshape(NC * NS, -1, K)
    ep16 = _ceil_to(ET, NS * K)
    src16 = jnp.concatenate(
        [src, jnp.zeros((ep16 - ET,), jnp.int32)]).reshape(NS, -1, K)
    dst16 = jnp.concatenate(
        [dst, jnp.full((ep16 - ET,), num_nodes, jnp.int32)]
    ).reshape(NS, -1, K)

    acc_rows = _acc_rows(num_nodes)
    deg = _sc_deg(dst32, num_nodes)
    g, dis = _tc_layer1(x, W1, deg, num_nodes)
    for (Wn, bp) in ((W2, b1), (W3, b2), (W4, b3)):
        msg = _sc_msg(g.reshape(NCH * num_nodes, CF), src16, dst16, num_nodes)
        g = _tc_layer(msg.reshape(NCH, acc_rows, CF), g, dis, bp, Wn, num_nodes)
    msg = _sc_msg(g.reshape(NCH * num_nodes, CF), src16, dst16, num_nodes)
    return _tc_head(msg.reshape(NCH, acc_rows, CF), g, dis, b4,
                    Wl1, bl1, Wl2, bl2, Wls, bls, B, N)


# speedup vs baseline: 8.1220x; 1.2134x over previous
import jax
import jax.numpy as jnp
from jax import lax
from jax.experimental import pallas as pl
from jax.experimental.pallas import tpu as pltpu
from jax.experimental.pallas import tpu_sc as plsc

NC = 2
NS = 16
L = 16
K = 128
CF = 16
NCH = 128 // CF
CPS = NCH // NC
NBUF = 4


def _leaky(v):
    return jnp.where(v >= 0, v, 0.01 * v)


def _ceil_to(x, m):
    return (x + m - 1) // m * m


def _fill_rows(ref, nrows, ncols, val, dtype):
    vec = jnp.full((L,), val, dtype)

    def body(i, _):
        for j in range(ncols // L):
            ref[i, pl.ds(j * L, L)] = vec
        return 0

    lax.fori_loop(0, nrows, body, 0)


def _acc_rows(num_nodes):
    return _ceil_to(num_nodes + K, NS * 8)


def _sc_deg(dst32, num_nodes):
    nb = dst32.shape[1]
    acc_rows = _acc_rows(num_nodes)
    zpt = acc_rows // NS
    mesh = plsc.VectorSubcoreMesh(core_axis_name="c", subcore_axis_name="s",
                                  num_cores=NC, num_subcores=NS)

    def body(dst_hbm, out_hbm, dst_v, ones_v, zbuf_v, acc_sh, *sems):
        cid = lax.axis_index("c")
        sid = lax.axis_index("s")
        wid = cid * NS + sid
        _fill_rows(ones_v, K, L, 1.0, jnp.float32)
        _fill_rows(zbuf_v, K, L, 0.0, jnp.float32)
        off = 0
        while off < zpt:
            step = min(K, zpt - off)
            pltpu.sync_copy(zbuf_v.at[pl.ds(0, step)],
                            acc_sh.at[pl.ds(sid * zpt + off, step)])
            off += step
        pltpu.sync_copy(dst_hbm.at[wid], dst_v)
        plsc.subcore_barrier()

        nq = nb // NBUF

        def quad(m, _):
            j0 = m * NBUF
            for b in range(NBUF):
                pltpu.async_copy(ones_v, acc_sh.at[dst_v.at[j0 + b]], sems[b],
                                 add=True)
            for b in range(NBUF):
                pltpu.make_async_copy(ones_v, acc_sh.at[dst_v.at[j0 + b]],
                                      sems[b]).wait()
            return 0

        lax.fori_loop(0, nq, quad, 0)
        for j in range(nq * NBUF, nb):
            pltpu.sync_copy(ones_v, acc_sh.at[dst_v.at[j]], add=True)
        plsc.subcore_barrier()
        pltpu.sync_copy(acc_sh.at[pl.ds(sid * zpt, zpt)],
                        out_hbm.at[cid, pl.ds(sid * zpt, zpt)])

    return pl.kernel(
        body,
        out_type=jax.ShapeDtypeStruct((NC, acc_rows, L), jnp.float32),
        mesh=mesh,
        scratch_types=[
            pltpu.VMEM((nb, K), jnp.int32),
            pltpu.VMEM((K, L), jnp.float32),
            pltpu.VMEM((K, L), jnp.float32),
            pltpu.VMEM_SHARED((acc_rows, L), jnp.float32),
        ] + [pltpu.SemaphoreType.DMA] * NBUF,
        compiler_params=pltpu.CompilerParams(use_tc_tiling_on_sc=False),
    )(dst32)


def _sc_msg(g8, src16, dst16, num_nodes):
    nb = src16.shape[1]
    nbp = nb + NBUF
    acc_rows = _acc_rows(num_nodes)
    zpt = acc_rows // NS
    mesh = plsc.VectorSubcoreMesh(core_axis_name="c", subcore_axis_name="s",
                                  num_cores=NC, num_subcores=NS)

    def body(g_hbm, src_hbm, dst_hbm, out_hbm, idx_v, dst_v, r0, r1, r2, r3,
             zbuf_v, acc_sh, sg0, sg1, sg2, sg3, ss0, ss1, ss2, ss3):
        rows = (r0, r1, r2, r3)
        sg = (sg0, sg1, sg2, sg3)
        ss = (ss0, ss1, ss2, ss3)
        cid = lax.axis_index("c")
        sid = lax.axis_index("s")
        _fill_rows(zbuf_v, K, CF, 0.0, jnp.float32)
        pltpu.sync_copy(src_hbm.at[sid], idx_v.at[pl.ds(0, nb)])
        _fill_rows(idx_v.at[pl.ds(nb, NBUF)], NBUF, K, 0, jnp.int32)
        pltpu.sync_copy(dst_hbm.at[sid], dst_v)

        def xform_idx(scale, delta):
            def sbody(i, _):
                for j in range(K // L):
                    sl = pl.ds(j * L, L)
                    idx_v[i, sl] = idx_v[i, sl] * scale + delta
                return 0
            lax.fori_loop(0, nbp, sbody, 0)

        for cc in range(CPS):
            if cc == 0:
                xform_idx(jnp.int32(8), cid * jnp.int32(CPS))
            else:
                xform_idx(jnp.int32(1), jnp.int32(1))
            col = (cid * CPS + cc) * CF

            def gather(j, b):
                return pltpu.async_copy(g_hbm.at[idx_v.at[j]], rows[b], sg[b])

            def gather_wait(j, b):
                pltpu.make_async_copy(g_hbm.at[idx_v.at[j]], rows[b],
                                      sg[b]).wait()

            def scatter(j, b):
                return pltpu.async_copy(rows[b], acc_sh.at[dst_v.at[j]], ss[b],
                                        add=True)

            def scatter_wait(j, b):
                pltpu.make_async_copy(rows[b], acc_sh.at[dst_v.at[j]],
                                      ss[b]).wait()

            off = 0
            while off < zpt:
                step = min(K, zpt - off)
                pltpu.sync_copy(zbuf_v.at[pl.ds(0, step)],
                                acc_sh.at[pl.ds(sid * zpt + off, step)])
                off += step
            plsc.subcore_barrier()

            for b in range(NBUF):
                gather(b, b)
            nq = nb // NBUF

            def quad(m, _):
                j0 = m * NBUF
                for b in range(NBUF):
                    gather_wait(j0 + b, b)
                    scatter(j0 + b, b)
                for b in range(NBUF):
                    scatter_wait(j0 + b, b)
                    gather(j0 + NBUF + b, b)
                return 0

            lax.fori_loop(0, nq, quad, 0)
            for j in range(nq * NBUF, nb):
                b = j % NBUF
                gather_wait(j, b)
                scatter(j, b)
                scatter_wait(j, b)
            for b in range(NBUF):
                j = nq * NBUF + b
                if j >= nb:
                    gather_wait(j, b)

            plsc.subcore_barrier()
            pltpu.sync_copy(acc_sh.at[pl.ds(sid * zpt, zpt)],
                            out_hbm.at[pl.ds(sid * zpt, zpt), pl.ds(col, CF)])
            plsc.subcore_barrier()

    return pl.kernel(
        body,
        out_type=jax.ShapeDtypeStruct((acc_rows, 128), jnp.float32),
        mesh=mesh,
        scratch_types=[
            pltpu.VMEM((nbp, K), jnp.int32),
            pltpu.VMEM((nb, K), jnp.int32),
            pltpu.VMEM((K, CF), jnp.float32),
            pltpu.VMEM((K, CF), jnp.float32),
            pltpu.VMEM((K, CF), jnp.float32),
            pltpu.VMEM((K, CF), jnp.float32),
            pltpu.VMEM((K, CF), jnp.float32),
            pltpu.VMEM_SHARED((acc_rows, CF), jnp.float32),
        ] + [pltpu.SemaphoreType.DMA] * (2 * NBUF),
        compiler_params=pltpu.CompilerParams(use_tc_tiling_on_sc=False),
    )(g8, src16, dst16)


_BR = 2000


def _tc_layer1(x, W1, deg, num_nodes):
    nblk = num_nodes // _BR

    def body(x_ref, w_ref, deg_ref, g_ref, dis_ref):
        d = deg_ref[...]
        deg_tot = d[0, :, 0:1] + d[1, :, 0:1] + 1.0
        dis = lax.rsqrt(deg_tot)
        dis_ref[...] = dis
        h = jnp.dot(x_ref[...], w_ref[...], preferred_element_type=jnp.float32)
        g_ref[...] = h * dis

    return pl.pallas_call(
        body,
        grid=(nblk,),
        in_specs=[
            pl.BlockSpec((_BR, 128), lambda i: (i, 0)),
            pl.BlockSpec((128, 128), lambda i: (0, 0)),
            pl.BlockSpec((NC, _BR, L), lambda i: (0, i, 0)),
        ],
        out_specs=[
            pl.BlockSpec((_BR, 128), lambda i: (i, 0)),
            pl.BlockSpec((_BR, 1), lambda i: (i, 0)),
        ],
        out_shape=[
            jax.ShapeDtypeStruct((num_nodes, 128), jnp.float32),
            jax.ShapeDtypeStruct((num_nodes, 1), jnp.float32),
        ],
    )(x, W1, deg)


def _tc_layer(msg, g, dis, b, W, num_nodes):
    nblk = num_nodes // _BR

    def body(msg_ref, g_ref, dis_ref, b_ref, w_ref, gn_ref):
        dis = dis_ref[...]
        y = _leaky(dis * (msg_ref[...] + g_ref[...]) + b_ref[...][None, :])
        gn_ref[...] = jnp.dot(y, w_ref[...],
                              preferred_element_type=jnp.float32) * dis

    return pl.pallas_call(
        body,
        grid=(nblk,),
        in_specs=[
            pl.BlockSpec((_BR, 128), lambda i: (i, 0)),
            pl.BlockSpec((_BR, 128), lambda i: (i, 0)),
            pl.BlockSpec((_BR, 1), lambda i: (i, 0)),
            pl.BlockSpec((128,), lambda i: (0,)),
            pl.BlockSpec((128, 128), lambda i: (0, 0)),
        ],
        out_specs=pl.BlockSpec((_BR, 128), lambda i: (i, 0)),
        out_shape=jax.ShapeDtypeStruct((num_nodes, 128), jnp.float32),
    )(msg, g, dis, b, W)


def _tc_head(msg, g, dis, b4, Wl1, bl1, Wl2, bl2, Wls, bls, batch, n):
    num_nodes = batch * n
    nblk = num_nodes // _BR
    bpg = n // _BR

    def body(msg_ref, g_ref, dis_ref, b_ref, wl1_ref, bl1_ref, wl2_ref,
             bl2_ref, wls_ref, bls_ref, out_ref, acc_ref):
        i = pl.program_id(0)

        @pl.when(i == 0)
        def _init():
            acc_ref[...] = jnp.zeros_like(acc_ref)

        dis = dis_ref[...]
        x = _leaky(dis * (msg_ref[...] + g_ref[...]) + b_ref[...][None, :])
        t = _leaky(jnp.dot(x, wl1_ref[...], preferred_element_type=jnp.float32)
                   + bl1_ref[...][None, :])
        u = jnp.dot(t, wl2_ref[...], preferred_element_type=jnp.float32) \
            + bl2_ref[...][None, :]
        usum = jnp.sum(u, axis=0, keepdims=True)
        gidx = i // bpg
        acc_ref[pl.ds(gidx, 1), :] += usum

        @pl.when(i == nblk - 1)
        def _fin():
            pooled = acc_ref[...]
            res = jnp.dot(pooled, wls_ref[...],
                          preferred_element_type=jnp.float32)
            out_ref[...] = res[:batch, :] + bls_ref[...][None, :]

    return pl.pallas_call(
        body,
        grid=(nblk,),
        in_specs=[
            pl.BlockSpec((_BR, 128), lambda i: (i, 0)),
            pl.BlockSpec((_BR, 128), lambda i: (i, 0)),
            pl.BlockSpec((_BR, 1), lambda i: (i, 0)),
            pl.BlockSpec((128,), lambda i: (0,)),
            pl.BlockSpec((128, 128), lambda i: (0, 0)),
            pl.BlockSpec((128,), lambda i: (0,)),
            pl.BlockSpec((128, 64), lambda i: (0, 0)),
            pl.BlockSpec((64,), lambda i: (0,)),
            pl.BlockSpec((64, 1), lambda i: (0, 0)),
            pl.BlockSpec((1,), lambda i: (0,)),
        ],
        out_specs=pl.BlockSpec((batch, 1), lambda i: (0, 0)),
        out_shape=jax.ShapeDtypeStruct((batch, 1), jnp.float32),
        scratch_shapes=[pltpu.VMEM((8, 64), jnp.float32)],
    )(msg, g, dis, b4, Wl1, bl1, Wl2, bl2, Wls, bls)


def kernel(graph_nodes, graph_edge_links, graph_edges, W1, b1, W2, b2, W3, b3,
           W4, b4, Wl1, bl1, Wl2, bl2, Wls, bls):
    B, N, D = graph_nodes.shape
    E = graph_edge_links.shape[2]
    num_nodes = B * N
    ET = B * E

    offsets = (jnp.arange(B, dtype=graph_edge_links.dtype) * N)[:, None, None]
    ei = (graph_edge_links + offsets).transpose(1, 0, 2).reshape(2, -1)
    src, dst = ei[0].astype(jnp.int32), ei[1].astype(jnp.int32)
    x = graph_nodes.reshape(num_nodes, D)

    ep32 = _ceil_to(ET, NC * NS * K)
    dst32 = jnp.concatenate(
        [dst, jnp.full((ep32 - ET,), num_nodes, jnp.int32)]
    ).reshape(NC * NS, -1, K)
    ep16 = _ceil_to(ET, NS * K)
    src16 = jnp.concatenate(
        [src, jnp.zeros((ep16 - ET,), jnp.int32)]).reshape(NS, -1, K)
    dst16 = jnp.concatenate(
        [dst, jnp.full((ep16 - ET,), num_nodes, jnp.int32)]
    ).reshape(NS, -1, K)

    acc_rows = _acc_rows(num_nodes)
    deg = _sc_deg(dst32, num_nodes)
    g, dis = _tc_layer1(x, W1, deg, num_nodes)
    for (Wn, bp) in ((W2, b1), (W3, b2), (W4, b3)):
        msg = _sc_msg(g.reshape(8 * num_nodes, L), src16, dst16, num_nodes)
        g = _tc_layer(msg, g, dis, bp, Wn, num_nodes)
    msg = _sc_msg(g.reshape(8 * num_nodes, L), src16, dst16, num_nodes)
    return _tc_head(msg, g, dis, b4,
                    Wl1, bl1, Wl2, bl2, Wls, bls, B, N)
